# Initial kernel scaffold; baseline (speedup 1.0000x reference)
import jax
import jax.numpy as jnp
from jax.experimental import pallas as pl


def kernel(x, table):
    raise NotImplementedError("write your pallas kernel here")


# speedup vs baseline: 1.3509x; 1.3509x over previous
import functools

import jax
import jax.numpy as jnp
from jax import lax
from jax.experimental import pallas as pl
from jax.experimental.pallas import tpu as pltpu
from jax.experimental.pallas import tpu_sc as plsc

NUM_CORES = 2
NUM_SUBCORES = 16
NUM_WORKERS = NUM_CORES * NUM_SUBCORES
CHUNK = 128
NBUF = 2


def _emb_kernel_body(n_chunks, d, x_hbm, table_hbm, out_hbm, idx_v, rows_v,
                     gsem):
  wid = lax.axis_index("s") * NUM_CORES + lax.axis_index("c")
  pltpu.sync_copy(x_hbm.at[wid], idx_v)

  pltpu.async_copy(table_hbm.at[idx_v.at[0]], rows_v.at[0], gsem)

  @pl.loop(0, n_chunks, step=NBUF)
  def _(g):
    for b in range(NBUF):
      j = g + b
      pltpu.make_async_copy(table_hbm.at[idx_v.at[j]], rows_v.at[b],
                            gsem).wait()
      nxt = j + 1

      @pl.when(nxt < n_chunks)
      def _():
        pltpu.async_copy(table_hbm.at[idx_v.at[nxt]],
                         rows_v.at[(b + 1) % NBUF], gsem)

      pltpu.sync_copy(rows_v.at[b], out_hbm.at[wid, j])


def kernel(x, table):
  b, h = x.shape
  _, d = table.shape
  n = b * h
  assert n % (NUM_WORKERS * CHUNK) == 0
  n_chunks = n // (NUM_WORKERS * CHUNK)

  x_flat = x.reshape(NUM_WORKERS, n_chunks, CHUNK).astype(jnp.int32)

  mesh = plsc.VectorSubcoreMesh(
      core_axis_name="c", subcore_axis_name="s", num_cores=NUM_CORES,
      num_subcores=NUM_SUBCORES)

  emb = pl.kernel(
      functools.partial(_emb_kernel_body, n_chunks, d),
      out_type=jax.ShapeDtypeStruct((NUM_WORKERS, n_chunks, CHUNK, d),
                                    jnp.float32),
      mesh=mesh,
      scratch_types=[
          pltpu.VMEM((n_chunks, CHUNK), jnp.int32),
          pltpu.VMEM((NBUF, CHUNK, d), jnp.float32),
          pltpu.SemaphoreType.DMA,
      ],
      compiler_params=pltpu.CompilerParams(use_tc_tiling_on_sc=False),
  )
  out = emb(x_flat, table)
  return out.reshape(b, h, d)


# speedup vs baseline: 1.5008x; 1.1109x over previous
import functools

import jax
import jax.numpy as jnp
from jax import lax
from jax.experimental import pallas as pl
from jax.experimental.pallas import tpu as pltpu
from jax.experimental.pallas import tpu_sc as plsc

NUM_CORES = 2
NUM_SUBCORES = 16
NUM_WORKERS = NUM_CORES * NUM_SUBCORES
CHUNK = 128
NBUF = 8


GDEPTH = NBUF - 2


def _emb_kernel_body(n_chunks, d, x_hbm, table_hbm, out_hbm, idx_v, rows_v,
                     gsem, ssem):
  wid = lax.axis_index("s") * NUM_CORES + lax.axis_index("c")
  pltpu.sync_copy(x_hbm.at[wid], idx_v)

  for b in range(GDEPTH):
    pltpu.async_copy(table_hbm.at[idx_v.at[b]], rows_v.at[b], gsem.at[b])

  @pl.loop(0, n_chunks, step=NBUF)
  def _(g):
    for b in range(NBUF):
      j = g + b
      pltpu.make_async_copy(table_hbm.at[idx_v.at[j]], rows_v.at[b],
                            gsem.at[b]).wait()
      pltpu.async_copy(rows_v.at[b], out_hbm.at[wid, j], ssem.at[b])

      nxt = j + GDEPTH
      b2 = (b + GDEPTH) % NBUF

      @pl.when(jnp.logical_and(nxt < n_chunks, nxt >= NBUF))
      def _():
        pltpu.make_async_copy(rows_v.at[b2], out_hbm.at[wid, nxt - NBUF],
                              ssem.at[b2]).wait()

      @pl.when(nxt < n_chunks)
      def _():
        pltpu.async_copy(table_hbm.at[idx_v.at[nxt]], rows_v.at[b2],
                         gsem.at[b2])

  for b in range(NBUF):
    pltpu.make_async_copy(rows_v.at[b], out_hbm.at[wid, n_chunks - NBUF + b],
                          ssem.at[b]).wait()


def kernel(x, table):
  b, h = x.shape
  _, d = table.shape
  n = b * h
  assert n % (NUM_WORKERS * CHUNK) == 0
  n_chunks = n // (NUM_WORKERS * CHUNK)

  x_flat = x.reshape(NUM_WORKERS, n_chunks, CHUNK).astype(jnp.int32)

  mesh = plsc.VectorSubcoreMesh(
      core_axis_name="c", subcore_axis_name="s", num_cores=NUM_CORES,
      num_subcores=NUM_SUBCORES)

  emb = pl.kernel(
      functools.partial(_emb_kernel_body, n_chunks, d),
      out_type=jax.ShapeDtypeStruct((NUM_WORKERS, n_chunks, CHUNK, d),
                                    jnp.float32),
      mesh=mesh,
      scratch_types=[
          pltpu.VMEM((n_chunks, CHUNK), jnp.int32),
          pltpu.VMEM((NBUF, CHUNK, d), jnp.float32),
          pltpu.SemaphoreType.DMA((NBUF,)),
          pltpu.SemaphoreType.DMA((NBUF,)),
      ],
      compiler_params=pltpu.CompilerParams(use_tc_tiling_on_sc=False),
  )
  out = emb(x_flat, table)
  return out.reshape(b, h, d)
